# Initial kernel scaffold; baseline (speedup 1.0000x reference)
import jax
import jax.numpy as jnp
from jax.experimental import pallas as pl


def kernel(x, y, sign1, indx1, sign2, indx2):
    raise NotImplementedError("write your pallas kernel here")


# speedup vs baseline: 1.5801x; 1.5801x over previous
import functools

import numpy as np
import jax
import jax.numpy as jnp
from jax.experimental import pallas as pl

N_OUT = 16000
N1 = 128
N2 = 125
D_IN = 2048
BATCH = 1024

_HI = jax.lax.Precision.HIGHEST


def _dft_constants():
    n2 = np.arange(N2)
    n1 = np.arange(N1)
    f125 = np.exp(-2j * np.pi * np.outer(n2, n2) / N2)
    f128 = np.exp(-2j * np.pi * np.outer(n1, n1) / N1)
    tw = np.exp(-2j * np.pi * np.outer(n2, n1) / N_OUT)
    g125 = np.conj(f125)
    g128 = np.conj(f128)

    f125_pack = np.concatenate([f125.real, f125.imag], axis=0)
    f128_pack = np.block([[f128.real, f128.imag],
                          [-f128.imag, f128.real]])
    g128_pack = np.block([[g128.real, g128.imag],
                          [-g128.imag, g128.real]])
    g125_pack = np.concatenate([g125.real, -g125.imag], axis=1)
    return (f125_pack.astype(np.float32), f128_pack.astype(np.float32),
            g128_pack.astype(np.float32), g125_pack.astype(np.float32),
            tw.real.astype(np.float32), tw.imag.astype(np.float32))


_F125P, _F128P, _G128P, _G125P, _TWR, _TWI = _dft_constants()


_BM = 512
_KT = 640


def _sketch_body(x_ref, indx_ref, sign_ref, out_ref):
    k0 = pl.program_id(1) * _KT
    idx = indx_ref[...]
    sgn = sign_ref[...]
    kk = jax.lax.broadcasted_iota(jnp.int32, (D_IN, _KT), 1) + k0
    s = jnp.where(idx == kk, sgn, jnp.float32(0.0))
    out_ref[...] = jnp.dot(x_ref[...], s, preferred_element_type=jnp.float32,
                           precision=_HI)


def _count_sketch(x, indx, sign):
    return pl.pallas_call(
        _sketch_body,
        grid=(BATCH // _BM, N_OUT // _KT),
        in_specs=[
            pl.BlockSpec((_BM, D_IN), lambda i, j: (i, 0)),
            pl.BlockSpec((D_IN, 1), lambda i, j: (0, 0)),
            pl.BlockSpec((D_IN, 1), lambda i, j: (0, 0)),
        ],
        out_specs=pl.BlockSpec((_BM, _KT), lambda i, j: (i, j)),
        out_shape=jax.ShapeDtypeStruct((BATCH, N_OUT), jnp.float32),
    )(x, indx.reshape(D_IN, 1), sign.reshape(D_IN, 1))


_BT = 16


def _fwd_spectrum(xt, f125p, f128p, twr, twi):
    x3 = xt.reshape(_BT, N2, N1).swapaxes(0, 1).reshape(N2, _BT * N1)
    c = jnp.dot(f125p, x3, preferred_element_type=jnp.float32, precision=_HI)
    cr = c[:N2].reshape(N2, _BT, N1)
    ci = c[N2:].reshape(N2, _BT, N1)
    twr3 = twr.reshape(N2, 1, N1)
    twi3 = twi.reshape(N2, 1, N1)
    dr = (cr * twr3 - ci * twi3).reshape(N2 * _BT, N1)
    di = (cr * twi3 + ci * twr3).reshape(N2 * _BT, N1)
    dpack = jnp.concatenate([dr, di], axis=1)
    return jnp.dot(dpack, f128p, preferred_element_type=jnp.float32,
                   precision=_HI)


def _fftconv_body(xcs_ref, ycs_ref, f125p_ref, f128p_ref, g128p_ref,
                  g125p_ref, twr_ref, twi_ref, out_ref):
    f125p = f125p_ref[...]
    f128p = f128p_ref[...]
    twr = twr_ref[...]
    twi = twi_ref[...]

    ex = _fwd_spectrum(xcs_ref[...], f125p, f128p, twr, twi)
    ey = _fwd_spectrum(ycs_ref[...], f125p, f128p, twr, twi)
    xr, xi = ex[:, :N1], ex[:, N1:]
    yr, yi = ey[:, :N1], ey[:, N1:]
    zr = xr * yr - xi * yi
    zi = xr * yi + xi * yr

    zpack = jnp.concatenate([zr, zi], axis=1)
    p = jnp.dot(zpack, g128p_ref[...], preferred_element_type=jnp.float32,
                precision=_HI)
    pr = p[:, :N1].reshape(N2, _BT, N1)
    pi = p[:, N1:].reshape(N2, _BT, N1)
    twr3 = twr.reshape(N2, 1, N1)
    twi3 = twi.reshape(N2, 1, N1)
    qr = (pr * twr3 + pi * twi3).reshape(N2, _BT * N1)
    qi = (pi * twr3 - pr * twi3).reshape(N2, _BT * N1)
    qpack = jnp.concatenate([qr, qi], axis=0)
    z3 = jnp.dot(g125p_ref[...], qpack, preferred_element_type=jnp.float32,
                 precision=_HI) * jnp.float32(1.0 / N_OUT)
    out_ref[...] = z3.reshape(N2, _BT, N1).swapaxes(0, 1).reshape(_BT, N_OUT)


def _fftconv(xcs, ycs):
    consts = (_F125P, _F128P, _G128P, _G125P, _TWR, _TWI)
    const_specs = [
        pl.BlockSpec(c.shape, functools.partial(lambda n, i: (0,) * n, c.ndim))
        for c in consts
    ]
    return pl.pallas_call(
        _fftconv_body,
        grid=(BATCH // _BT,),
        in_specs=[
            pl.BlockSpec((_BT, N_OUT), lambda i: (i, 0)),
            pl.BlockSpec((_BT, N_OUT), lambda i: (i, 0)),
            *const_specs,
        ],
        out_specs=pl.BlockSpec((_BT, N_OUT), lambda i: (i, 0)),
        out_shape=jax.ShapeDtypeStruct((BATCH, N_OUT), jnp.float32),
    )(xcs, ycs, *(jnp.asarray(c) for c in consts))


@jax.jit
def kernel(x, y, sign1, indx1, sign2, indx2):
    xcs = _count_sketch(x, indx1, sign1)
    ycs = _count_sketch(y, indx2, sign2)
    return _fftconv(xcs, ycs)


# speedup vs baseline: 4.7818x; 3.0263x over previous
import functools

import numpy as np
import jax
import jax.numpy as jnp
from jax.experimental import pallas as pl

N_OUT = 16000
N1 = 128
N2 = 125
D_IN = 2048
BATCH = 1024

_HI = jax.lax.Precision.HIGHEST


def _dft_constants():
    n2 = np.arange(N2)
    n1 = np.arange(N1)
    f125 = np.exp(-2j * np.pi * np.outer(n2, n2) / N2)
    f128 = np.exp(-2j * np.pi * np.outer(n1, n1) / N1)
    tw = np.exp(-2j * np.pi * np.outer(n2, n1) / N_OUT)
    g125 = np.conj(f125)
    g128 = np.conj(f128)

    f125_pack = np.concatenate([f125.real, f125.imag], axis=0)
    f128_pack = np.block([[f128.real, f128.imag],
                          [-f128.imag, f128.real]])
    g128_pack = np.block([[g128.real, g128.imag],
                          [-g128.imag, g128.real]])
    g125_pack = np.concatenate([g125.real, -g125.imag], axis=1)
    bf16 = jnp.bfloat16
    return (f125_pack.astype(bf16), f128_pack.astype(bf16),
            g128_pack.astype(bf16), g125_pack.astype(bf16),
            tw.real.astype(np.float32), tw.imag.astype(np.float32))


_F125P, _F128P, _G128P, _G125P, _TWR, _TWI = _dft_constants()


_BM = 512
_KT = 640


def _sketch_body(x_ref, indx_ref, sign_ref, out_ref):
    k0 = pl.program_id(1) * _KT
    idx = indx_ref[...]
    sgn = sign_ref[...]
    kk = jax.lax.broadcasted_iota(jnp.int32, (D_IN, _KT), 1) + k0
    s = jnp.where(idx == kk, sgn, jnp.float32(0.0)).astype(jnp.bfloat16)
    xf = x_ref[...]
    xhi = xf.astype(jnp.bfloat16)
    xlo = (xf - xhi.astype(jnp.float32)).astype(jnp.bfloat16)
    out_ref[...] = (
        jnp.dot(xhi, s, preferred_element_type=jnp.float32)
        + jnp.dot(xlo, s, preferred_element_type=jnp.float32))


def _count_sketch(x, indx, sign):
    return pl.pallas_call(
        _sketch_body,
        grid=(BATCH // _BM, N_OUT // _KT),
        in_specs=[
            pl.BlockSpec((_BM, D_IN), lambda i, j: (i, 0)),
            pl.BlockSpec((D_IN, 1), lambda i, j: (0, 0)),
            pl.BlockSpec((D_IN, 1), lambda i, j: (0, 0)),
        ],
        out_specs=pl.BlockSpec((_BM, _KT), lambda i, j: (i, j)),
        out_shape=jax.ShapeDtypeStruct((BATCH, N_OUT), jnp.float32),
    )(x, indx.reshape(D_IN, 1), sign.reshape(D_IN, 1))


_BT = 16


def _fwd_spectrum(xt, f125p, f128p, twr, twi):
    x3 = xt.reshape(_BT, N2, N1).swapaxes(0, 1).reshape(N2, _BT * N1)
    c = jnp.dot(f125p, x3.astype(jnp.bfloat16),
                preferred_element_type=jnp.float32)
    cr = c[:N2].reshape(N2, _BT, N1)
    ci = c[N2:].reshape(N2, _BT, N1)
    twr3 = twr.reshape(N2, 1, N1)
    twi3 = twi.reshape(N2, 1, N1)
    dr = (cr * twr3 - ci * twi3).reshape(N2 * _BT, N1)
    di = (cr * twi3 + ci * twr3).reshape(N2 * _BT, N1)
    dpack = jnp.concatenate([dr, di], axis=1).astype(jnp.bfloat16)
    return jnp.dot(dpack, f128p, preferred_element_type=jnp.float32)


def _fftconv_body(xcs_ref, ycs_ref, f125p_ref, f128p_ref, g128p_ref,
                  g125p_ref, twr_ref, twi_ref, out_ref):
    f125p = f125p_ref[...]
    f128p = f128p_ref[...]
    twr = twr_ref[...]
    twi = twi_ref[...]

    ex = _fwd_spectrum(xcs_ref[...], f125p, f128p, twr, twi)
    ey = _fwd_spectrum(ycs_ref[...], f125p, f128p, twr, twi)
    xr, xi = ex[:, :N1], ex[:, N1:]
    yr, yi = ey[:, :N1], ey[:, N1:]
    zr = xr * yr - xi * yi
    zi = xr * yi + xi * yr

    zpack = jnp.concatenate([zr, zi], axis=1).astype(jnp.bfloat16)
    p = jnp.dot(zpack, g128p_ref[...], preferred_element_type=jnp.float32)
    pr = p[:, :N1].reshape(N2, _BT, N1)
    pi = p[:, N1:].reshape(N2, _BT, N1)
    twr3 = twr.reshape(N2, 1, N1)
    twi3 = twi.reshape(N2, 1, N1)
    qr = (pr * twr3 + pi * twi3).reshape(N2, _BT * N1)
    qi = (pi * twr3 - pr * twi3).reshape(N2, _BT * N1)
    qpack = jnp.concatenate([qr, qi], axis=0).astype(jnp.bfloat16)
    z3 = jnp.dot(g125p_ref[...], qpack,
                 preferred_element_type=jnp.float32) * jnp.float32(1.0 / N_OUT)
    out_ref[...] = z3.reshape(N2, _BT, N1).swapaxes(0, 1).reshape(_BT, N_OUT)


def _fftconv(xcs, ycs):
    consts = (_F125P, _F128P, _G128P, _G125P, _TWR, _TWI)
    const_specs = [
        pl.BlockSpec(c.shape, functools.partial(lambda n, i: (0,) * n, c.ndim))
        for c in consts
    ]
    return pl.pallas_call(
        _fftconv_body,
        grid=(BATCH // _BT,),
        in_specs=[
            pl.BlockSpec((_BT, N_OUT), lambda i: (i, 0)),
            pl.BlockSpec((_BT, N_OUT), lambda i: (i, 0)),
            *const_specs,
        ],
        out_specs=pl.BlockSpec((_BT, N_OUT), lambda i: (i, 0)),
        out_shape=jax.ShapeDtypeStruct((BATCH, N_OUT), jnp.float32),
    )(xcs, ycs, *(jnp.asarray(c) for c in consts))


@jax.jit
def kernel(x, y, sign1, indx1, sign2, indx2):
    xcs = _count_sketch(x, indx1, sign1)
    ycs = _count_sketch(y, indx2, sign2)
    return _fftconv(xcs, ycs)


# speedup vs baseline: 4.9113x; 1.0271x over previous
import functools

import numpy as np
import jax
from jax import lax
import jax.numpy as jnp
from jax.experimental import pallas as pl
from jax.experimental.pallas import tpu as pltpu
from jax.experimental.pallas import tpu_sc as plsc

N_OUT = 16000
N1 = 128
N2 = 125
D_IN = 2048
BATCH = 1024

_HI = jax.lax.Precision.HIGHEST


def _dft_constants():
    n2 = np.arange(N2)
    n1 = np.arange(N1)
    f125 = np.exp(-2j * np.pi * np.outer(n2, n2) / N2)
    f128 = np.exp(-2j * np.pi * np.outer(n1, n1) / N1)
    tw = np.exp(-2j * np.pi * np.outer(n2, n1) / N_OUT)
    g125 = np.conj(f125)
    g128 = np.conj(f128)

    f125_pack = np.concatenate([f125.real, f125.imag], axis=0)
    f128_pack = np.block([[f128.real, f128.imag],
                          [-f128.imag, f128.real]])
    g128_pack = np.block([[g128.real, g128.imag],
                          [-g128.imag, g128.real]])
    g125_pack = np.concatenate([g125.real, -g125.imag], axis=1)
    bf16 = jnp.bfloat16
    return (f125_pack.astype(bf16), f128_pack.astype(bf16),
            g128_pack.astype(bf16), g125_pack.astype(bf16),
            tw.real.astype(np.float32), tw.imag.astype(np.float32))


_F125P, _F128P, _G128P, _G125P, _TWR, _TWI = _dft_constants()


_BM = 512
_KT = 640


def _sketch_body(x_ref, indx_ref, sign_ref, out_ref):
    k0 = pl.program_id(1) * _KT
    idx = indx_ref[...]
    sgn = sign_ref[...]
    kk = jax.lax.broadcasted_iota(jnp.int32, (D_IN, _KT), 1) + k0
    s = jnp.where(idx == kk, sgn, jnp.float32(0.0)).astype(jnp.bfloat16)
    xf = x_ref[...]
    xhi = xf.astype(jnp.bfloat16)
    xlo = (xf - xhi.astype(jnp.float32)).astype(jnp.bfloat16)
    out_ref[...] = (
        jnp.dot(xhi, s, preferred_element_type=jnp.float32)
        + jnp.dot(xlo, s, preferred_element_type=jnp.float32))


def _count_sketch(x, indx, sign):
    return pl.pallas_call(
        _sketch_body,
        grid=(BATCH // _BM, N_OUT // _KT),
        in_specs=[
            pl.BlockSpec((_BM, D_IN), lambda i, j: (i, 0)),
            pl.BlockSpec((D_IN, 1), lambda i, j: (0, 0)),
            pl.BlockSpec((D_IN, 1), lambda i, j: (0, 0)),
        ],
        out_specs=pl.BlockSpec((_BM, _KT), lambda i, j: (i, j)),
        out_shape=jax.ShapeDtypeStruct((BATCH, N_OUT), jnp.float32),
    )(x, indx.reshape(D_IN, 1), sign.reshape(D_IN, 1))


_BT = 16


def _fwd_spectrum(xt, f125p, f128p, twr, twi):
    x3 = xt.reshape(_BT, N2, N1).swapaxes(0, 1).reshape(N2, _BT * N1)
    c = jnp.dot(f125p, x3.astype(jnp.bfloat16),
                preferred_element_type=jnp.float32)
    cr = c[:N2].reshape(N2, _BT, N1)
    ci = c[N2:].reshape(N2, _BT, N1)
    twr3 = twr.reshape(N2, 1, N1)
    twi3 = twi.reshape(N2, 1, N1)
    dr = (cr * twr3 - ci * twi3).reshape(N2 * _BT, N1)
    di = (cr * twi3 + ci * twr3).reshape(N2 * _BT, N1)
    dpack = jnp.concatenate([dr, di], axis=1).astype(jnp.bfloat16)
    return jnp.dot(dpack, f128p, preferred_element_type=jnp.float32)


def _fftconv_body(xcs_ref, ycs_ref, f125p_ref, f128p_ref, g128p_ref,
                  g125p_ref, twr_ref, twi_ref, out_ref):
    f125p = f125p_ref[...]
    f128p = f128p_ref[...]
    twr = twr_ref[...]
    twi = twi_ref[...]

    ex = _fwd_spectrum(xcs_ref[...], f125p, f128p, twr, twi)
    ey = _fwd_spectrum(ycs_ref[...], f125p, f128p, twr, twi)
    xr, xi = ex[:, :N1], ex[:, N1:]
    yr, yi = ey[:, :N1], ey[:, N1:]
    zr = xr * yr - xi * yi
    zi = xr * yi + xi * yr

    zpack = jnp.concatenate([zr, zi], axis=1).astype(jnp.bfloat16)
    p = jnp.dot(zpack, g128p_ref[...], preferred_element_type=jnp.float32)
    pr = p[:, :N1].reshape(N2, _BT, N1)
    pi = p[:, N1:].reshape(N2, _BT, N1)
    twr3 = twr.reshape(N2, 1, N1)
    twi3 = twi.reshape(N2, 1, N1)
    qr = (pr * twr3 + pi * twi3).reshape(N2, _BT * N1)
    qi = (pi * twr3 - pr * twi3).reshape(N2, _BT * N1)
    qpack = jnp.concatenate([qr, qi], axis=0).astype(jnp.bfloat16)
    z3 = jnp.dot(g125p_ref[...], qpack,
                 preferred_element_type=jnp.float32) * jnp.float32(1.0 / N_OUT)
    out_ref[...] = z3.reshape(N2, _BT, N1).swapaxes(0, 1).reshape(_BT, N_OUT)


def _fftconv(xcs, ycs):
    consts = (_F125P, _F128P, _G128P, _G125P, _TWR, _TWI)
    const_specs = [
        pl.BlockSpec(c.shape, functools.partial(lambda n, i: (0,) * n, c.ndim))
        for c in consts
    ]
    return pl.pallas_call(
        _fftconv_body,
        grid=(BATCH // _BT,),
        in_specs=[
            pl.BlockSpec((_BT, N_OUT), lambda i: (i, 0)),
            pl.BlockSpec((_BT, N_OUT), lambda i: (i, 0)),
            *const_specs,
        ],
        out_specs=pl.BlockSpec((_BT, N_OUT), lambda i: (i, 0)),
        out_shape=jax.ShapeDtypeStruct((BATCH, N_OUT), jnp.float32),
    )(xcs, ycs, *(jnp.asarray(c) for c in consts))


_NW = 32
_ROWS_PER_W = BATCH // _NW
_NCHUNK = D_IN // 128


def _sc_sketch_pair(x, y, sign1, indx1, sign2, indx2):
    mesh = plsc.VectorSubcoreMesh(core_axis_name="c", subcore_axis_name="s")
    out_sds = jax.ShapeDtypeStruct((BATCH, N_OUT), jnp.float32)

    @functools.partial(
        pl.kernel,
        out_type=(out_sds, out_sds),
        mesh=mesh,
        scratch_types=[
            pltpu.VMEM((D_IN,), jnp.float32),
            pltpu.VMEM((D_IN,), jnp.float32),
            pltpu.VMEM((D_IN,), jnp.float32),
            pltpu.VMEM((_NCHUNK, 128), jnp.int32),
            pltpu.VMEM((_NCHUNK, 128), jnp.int32),
            pltpu.VMEM((128,), jnp.float32),
            pltpu.VMEM_SHARED((16 * N_OUT,), jnp.float32),
        ],
    )
    def sketch(x_hbm, y_hbm, s1_hbm, s2_hbm, i1_hbm, i2_hbm,
               xcs_hbm, ycs_hbm, xs_v, s1_v, s2_v, i1_v, i2_v, z_v, acc_sh):
        sid = lax.axis_index("s")
        wid = sid * 2 + lax.axis_index("c")
        base = wid * _ROWS_PER_W
        abase = sid * N_OUT

        pltpu.sync_copy(s1_hbm, s1_v)
        pltpu.sync_copy(s2_hbm, s2_v)
        pltpu.sync_copy(i1_hbm, i1_v)
        pltpu.sync_copy(i2_hbm, i2_v)

        @pl.loop(0, _NCHUNK)
        def _(j):
            @pl.loop(0, 128, step=16)
            def _(c):
                i1_v[j, pl.ds(c, 16)] = i1_v[j, pl.ds(c, 16)] + abase
                i2_v[j, pl.ds(c, 16)] = i2_v[j, pl.ds(c, 16)] + abase

        @pl.loop(0, 128, step=16)
        def _(c):
            z_v[pl.ds(c, 16)] = jnp.zeros((16,), jnp.float32)

        @pl.loop(0, D_IN, step=16)
        def _(c):
            xs_v[pl.ds(c, 16)] = jnp.zeros((16,), jnp.float32)

        @pl.loop(0, N_OUT, step=D_IN)
        def _(c):
            pltpu.sync_copy(xs_v, acc_sh.at[pl.ds(abase + c, D_IN)])

        def one_sketch(row_hbm, sgn_v, idx_v, out_hbm, b):
            pltpu.sync_copy(row_hbm.at[b], xs_v)

            @pl.loop(0, D_IN, step=16)
            def _(c):
                xs_v[pl.ds(c, 16)] = xs_v[pl.ds(c, 16)] * sgn_v[pl.ds(c, 16)]

            @pl.loop(0, _NCHUNK)
            def _(j):
                pltpu.sync_copy(xs_v.at[pl.ds(j * 128, 128)],
                                acc_sh.at[idx_v.at[j]], add=True)

            pltpu.sync_copy(acc_sh.at[pl.ds(abase, N_OUT)], out_hbm.at[b])

            @pl.loop(0, _NCHUNK)
            def _(j):
                pltpu.sync_copy(z_v, acc_sh.at[idx_v.at[j]])

        @pl.loop(0, _ROWS_PER_W)
        def _(i):
            b = base + i
            one_sketch(x_hbm, s1_v, i1_v, xcs_hbm, b)
            one_sketch(y_hbm, s2_v, i2_v, ycs_hbm, b)

    return sketch(x, y, sign1, sign2,
                  indx1.reshape(_NCHUNK, 128), indx2.reshape(_NCHUNK, 128))


@jax.jit
def kernel(x, y, sign1, indx1, sign2, indx2):
    xcs, ycs = _sc_sketch_pair(x, y, sign1, indx1, sign2, indx2)
    return _fftconv(xcs, ycs)


# speedup vs baseline: 6.1217x; 1.2465x over previous
import functools

import numpy as np
import jax
from jax import lax
import jax.numpy as jnp
from jax.experimental import pallas as pl
from jax.experimental.pallas import tpu as pltpu
from jax.experimental.pallas import tpu_sc as plsc

N_OUT = 16000
N1 = 128
N2 = 125
D_IN = 2048
BATCH = 1024

_HI = jax.lax.Precision.HIGHEST


def _dft_constants():
    n2 = np.arange(N2)
    n1 = np.arange(N1)
    f125 = np.exp(-2j * np.pi * np.outer(n2, n2) / N2)
    f128 = np.exp(-2j * np.pi * np.outer(n1, n1) / N1)
    tw = np.exp(-2j * np.pi * np.outer(n2, n1) / N_OUT)
    g125 = np.conj(f125)
    g128 = np.conj(f128)

    f125_pack = np.concatenate([f125.real, f125.imag], axis=0)
    f128_pack = np.block([[f128.real, f128.imag],
                          [-f128.imag, f128.real]])
    g128_pack = np.block([[g128.real, g128.imag],
                          [-g128.imag, g128.real]])
    g125_pack = np.concatenate([g125.real, -g125.imag], axis=1)
    bf16 = jnp.bfloat16
    return (f125_pack.astype(bf16), f128_pack.astype(bf16),
            g128_pack.astype(bf16), g125_pack.astype(bf16),
            tw.real.astype(np.float32), tw.imag.astype(np.float32))


_F125P, _F128P, _G128P, _G125P, _TWR, _TWI = _dft_constants()


_BM = 512
_KT = 640


def _sketch_body(x_ref, indx_ref, sign_ref, out_ref):
    k0 = pl.program_id(1) * _KT
    idx = indx_ref[...]
    sgn = sign_ref[...]
    kk = jax.lax.broadcasted_iota(jnp.int32, (D_IN, _KT), 1) + k0
    s = jnp.where(idx == kk, sgn, jnp.float32(0.0)).astype(jnp.bfloat16)
    xf = x_ref[...]
    xhi = xf.astype(jnp.bfloat16)
    xlo = (xf - xhi.astype(jnp.float32)).astype(jnp.bfloat16)
    out_ref[...] = (
        jnp.dot(xhi, s, preferred_element_type=jnp.float32)
        + jnp.dot(xlo, s, preferred_element_type=jnp.float32))


def _count_sketch(x, indx, sign):
    return pl.pallas_call(
        _sketch_body,
        grid=(BATCH // _BM, N_OUT // _KT),
        in_specs=[
            pl.BlockSpec((_BM, D_IN), lambda i, j: (i, 0)),
            pl.BlockSpec((D_IN, 1), lambda i, j: (0, 0)),
            pl.BlockSpec((D_IN, 1), lambda i, j: (0, 0)),
        ],
        out_specs=pl.BlockSpec((_BM, _KT), lambda i, j: (i, j)),
        out_shape=jax.ShapeDtypeStruct((BATCH, N_OUT), jnp.float32),
    )(x, indx.reshape(D_IN, 1), sign.reshape(D_IN, 1))


_BT = 16


def _fftconv_body(xcs_ref, ycs_ref, f125p_ref, f128p_ref, g128p_ref,
                  g125p_ref, twr_ref, twi_ref, out_ref):
    bf16 = jnp.bfloat16
    f125p = f125p_ref[...]
    f128p = f128p_ref[...]
    twr3 = twr_ref[...].reshape(N2, 1, N1)
    twi3 = twi_ref[...].reshape(N2, 1, N1)

    x3 = (xcs_ref[...].astype(bf16).reshape(_BT, N2, N1)
          .swapaxes(0, 1).reshape(N2, _BT * N1))
    y3 = (ycs_ref[...].astype(bf16).reshape(_BT, N2, N1)
          .swapaxes(0, 1).reshape(N2, _BT * N1))
    c = jnp.dot(f125p, jnp.concatenate([x3, y3], axis=1),
                preferred_element_type=jnp.float32)
    cb = c.astype(bf16).reshape(2, N2, 2 * _BT, N1)
    cxr, cxi = cb[0, :, :_BT], cb[1, :, :_BT]
    cyr, cyi = cb[0, :, _BT:], cb[1, :, _BT:]
    dxr = (cxr * twr3 - cxi * twi3).reshape(N2 * _BT, N1)
    dxi = (cxr * twi3 + cxi * twr3).reshape(N2 * _BT, N1)
    dyr = (cyr * twr3 - cyi * twi3).reshape(N2 * _BT, N1)
    dyi = (cyr * twi3 + cyi * twr3).reshape(N2 * _BT, N1)
    dpack = jnp.concatenate(
        [jnp.concatenate([dxr, dxi], axis=1),
         jnp.concatenate([dyr, dyi], axis=1)], axis=0)
    e = jnp.dot(dpack, f128p, preferred_element_type=jnp.float32)

    m = N2 * _BT
    eb = e.astype(bf16)
    xr, xi = eb[:m, :N1], eb[:m, N1:]
    yr, yi = eb[m:, :N1], eb[m:, N1:]
    zr = xr * yr - xi * yi
    zi = xr * yi + xi * yr

    zpack = jnp.concatenate([zr, zi], axis=1)
    p = jnp.dot(zpack, g128p_ref[...], preferred_element_type=jnp.float32)
    pb = p.astype(bf16).reshape(N2, _BT, 2 * N1)
    pr, pi = pb[:, :, :N1], pb[:, :, N1:]
    qr = (pr * twr3 + pi * twi3).reshape(N2, _BT * N1)
    qi = (pi * twr3 - pr * twi3).reshape(N2, _BT * N1)
    qpack = jnp.concatenate([qr, qi], axis=0)
    z3 = jnp.dot(g125p_ref[...], qpack,
                 preferred_element_type=jnp.float32) * jnp.float32(1.0 / N_OUT)
    out_ref[...] = z3.reshape(N2, _BT, N1).swapaxes(0, 1).reshape(_BT, N_OUT)


def _fftconv(xcs, ycs):
    nb = xcs.shape[0]
    consts = (_F125P, _F128P, _G128P, _G125P,
              _TWR.astype(jnp.bfloat16), _TWI.astype(jnp.bfloat16))
    const_specs = [
        pl.BlockSpec(c.shape, functools.partial(lambda n, i: (0,) * n, c.ndim))
        for c in consts
    ]
    return pl.pallas_call(
        _fftconv_body,
        grid=(nb // _BT,),
        in_specs=[
            pl.BlockSpec((_BT, N_OUT), lambda i: (i, 0)),
            pl.BlockSpec((_BT, N_OUT), lambda i: (i, 0)),
            *const_specs,
        ],
        out_specs=pl.BlockSpec((_BT, N_OUT), lambda i: (i, 0)),
        out_shape=jax.ShapeDtypeStruct((nb, N_OUT), jnp.float32),
    )(xcs, ycs, *(jnp.asarray(c) for c in consts))


_NW = 32
_NCHUNK = D_IN // 128


def _sc_sketch_pair(x, y, sign1, indx1, sign2, indx2):
    nb = x.shape[0]
    rows_per_w = nb // _NW
    mesh = plsc.VectorSubcoreMesh(core_axis_name="c", subcore_axis_name="s")
    out_sds = jax.ShapeDtypeStruct((nb, N_OUT), jnp.float32)

    @functools.partial(
        pl.kernel,
        out_type=(out_sds, out_sds),
        mesh=mesh,
        scratch_types=[
            pltpu.VMEM((D_IN,), jnp.float32),
            pltpu.VMEM((D_IN,), jnp.float32),
            pltpu.VMEM((D_IN,), jnp.float32),
            pltpu.VMEM((_NCHUNK, 128), jnp.int32),
            pltpu.VMEM((_NCHUNK, 128), jnp.int32),
            pltpu.VMEM((128,), jnp.float32),
            pltpu.VMEM_SHARED((16 * N_OUT,), jnp.float32),
        ],
    )
    def sketch(x_hbm, y_hbm, s1_hbm, s2_hbm, i1_hbm, i2_hbm,
               xcs_hbm, ycs_hbm, xs_v, s1_v, s2_v, i1_v, i2_v, z_v, acc_sh):
        sid = lax.axis_index("s")
        wid = sid * 2 + lax.axis_index("c")
        base = wid * rows_per_w
        abase = sid * N_OUT

        pltpu.sync_copy(s1_hbm, s1_v)
        pltpu.sync_copy(s2_hbm, s2_v)
        pltpu.sync_copy(i1_hbm, i1_v)
        pltpu.sync_copy(i2_hbm, i2_v)

        @pl.loop(0, _NCHUNK)
        def _(j):
            @pl.loop(0, 128, step=16)
            def _(c):
                i1_v[j, pl.ds(c, 16)] = i1_v[j, pl.ds(c, 16)] + abase
                i2_v[j, pl.ds(c, 16)] = i2_v[j, pl.ds(c, 16)] + abase

        @pl.loop(0, 128, step=16)
        def _(c):
            z_v[pl.ds(c, 16)] = jnp.zeros((16,), jnp.float32)

        @pl.loop(0, D_IN, step=16)
        def _(c):
            xs_v[pl.ds(c, 16)] = jnp.zeros((16,), jnp.float32)

        @pl.loop(0, N_OUT, step=D_IN)
        def _(c):
            pltpu.sync_copy(xs_v, acc_sh.at[pl.ds(abase + c, D_IN)])

        def one_sketch(row_hbm, sgn_v, idx_v, out_hbm, b):
            pltpu.sync_copy(row_hbm.at[b], xs_v)

            @pl.loop(0, D_IN, step=16)
            def _(c):
                xs_v[pl.ds(c, 16)] = xs_v[pl.ds(c, 16)] * sgn_v[pl.ds(c, 16)]

            @pl.loop(0, _NCHUNK)
            def _(j):
                pltpu.sync_copy(xs_v.at[pl.ds(j * 128, 128)],
                                acc_sh.at[idx_v.at[j]], add=True)

            pltpu.sync_copy(acc_sh.at[pl.ds(abase, N_OUT)], out_hbm.at[b])

            @pl.loop(0, _NCHUNK)
            def _(j):
                pltpu.sync_copy(z_v, acc_sh.at[idx_v.at[j]])

        @pl.loop(0, rows_per_w)
        def _(i):
            b = base + i
            one_sketch(x_hbm, s1_v, i1_v, xcs_hbm, b)
            one_sketch(y_hbm, s2_v, i2_v, ycs_hbm, b)

    return sketch(x, y, sign1, sign2,
                  indx1.reshape(_NCHUNK, 128), indx2.reshape(_NCHUNK, 128))


_NCHUNKS_B = 4


@jax.jit
def kernel(x, y, sign1, indx1, sign2, indx2):
    cb = BATCH // _NCHUNKS_B
    outs = []
    for c in range(_NCHUNKS_B):
        sl = slice(c * cb, (c + 1) * cb)
        xcs, ycs = _sc_sketch_pair(x[sl], y[sl], sign1, indx1, sign2, indx2)
        outs.append(_fftconv(xcs, ycs))
    return jnp.concatenate(outs, axis=0)


# speedup vs baseline: 6.8267x; 1.1152x over previous
import functools

import numpy as np
import jax
from jax import lax
import jax.numpy as jnp
from jax.experimental import pallas as pl
from jax.experimental.pallas import tpu as pltpu
from jax.experimental.pallas import tpu_sc as plsc

N_OUT = 16000
N1 = 128
N2 = 125
D_IN = 2048
BATCH = 1024

_HI = jax.lax.Precision.HIGHEST


def _dft_constants():
    n2 = np.arange(N2)
    n1 = np.arange(N1)
    f125 = np.exp(-2j * np.pi * np.outer(n2, n2) / N2)
    f128 = np.exp(-2j * np.pi * np.outer(n1, n1) / N1)
    tw = np.exp(-2j * np.pi * np.outer(n2, n1) / N_OUT)
    g125 = np.conj(f125)
    g128 = np.conj(f128)

    f125_pack = np.concatenate([f125.real, f125.imag], axis=0)
    f128_pack = np.block([[f128.real, f128.imag],
                          [-f128.imag, f128.real]])
    g128_pack = np.block([[g128.real, g128.imag],
                          [-g128.imag, g128.real]])
    g125_pack = np.concatenate([g125.real, -g125.imag], axis=1)
    bf16 = jnp.bfloat16
    return (f125_pack.astype(bf16), f128_pack.astype(bf16),
            g128_pack.astype(bf16), g125_pack.astype(bf16),
            tw.real.astype(np.float32), tw.imag.astype(np.float32))


_F125P, _F128P, _G128P, _G125P, _TWR, _TWI = _dft_constants()


_BM = 512
_KT = 640


def _sketch_body(x_ref, indx_ref, sign_ref, out_ref):
    k0 = pl.program_id(1) * _KT
    idx = indx_ref[...]
    sgn = sign_ref[...]
    kk = jax.lax.broadcasted_iota(jnp.int32, (D_IN, _KT), 1) + k0
    s = jnp.where(idx == kk, sgn, jnp.float32(0.0)).astype(jnp.bfloat16)
    xf = x_ref[...]
    xhi = xf.astype(jnp.bfloat16)
    xlo = (xf - xhi.astype(jnp.float32)).astype(jnp.bfloat16)
    out_ref[...] = (
        jnp.dot(xhi, s, preferred_element_type=jnp.float32)
        + jnp.dot(xlo, s, preferred_element_type=jnp.float32))


def _count_sketch(x, indx, sign):
    return pl.pallas_call(
        _sketch_body,
        grid=(BATCH // _BM, N_OUT // _KT),
        in_specs=[
            pl.BlockSpec((_BM, D_IN), lambda i, j: (i, 0)),
            pl.BlockSpec((D_IN, 1), lambda i, j: (0, 0)),
            pl.BlockSpec((D_IN, 1), lambda i, j: (0, 0)),
        ],
        out_specs=pl.BlockSpec((_BM, _KT), lambda i, j: (i, j)),
        out_shape=jax.ShapeDtypeStruct((BATCH, N_OUT), jnp.float32),
    )(x, indx.reshape(D_IN, 1), sign.reshape(D_IN, 1))


_BT = 16


def _fftconv_body(xcs_ref, ycs_ref, f125p_ref, f128p_ref, g128p_ref,
                  g125p_ref, twr_ref, twi_ref, out_ref):
    bf16 = jnp.bfloat16
    f125p = f125p_ref[...]
    f128p = f128p_ref[...]
    twr3 = twr_ref[...].reshape(N2, 1, N1)
    twi3 = twi_ref[...].reshape(N2, 1, N1)

    x3 = (xcs_ref[...].astype(bf16).reshape(_BT, N2, N1)
          .swapaxes(0, 1).reshape(N2, _BT * N1))
    y3 = (ycs_ref[...].astype(bf16).reshape(_BT, N2, N1)
          .swapaxes(0, 1).reshape(N2, _BT * N1))
    c = jnp.dot(f125p, jnp.concatenate([x3, y3], axis=1),
                preferred_element_type=jnp.float32)
    cb = c.astype(bf16).reshape(2, N2, 2 * _BT, N1)
    cxr, cxi = cb[0, :, :_BT], cb[1, :, :_BT]
    cyr, cyi = cb[0, :, _BT:], cb[1, :, _BT:]
    dxr = (cxr * twr3 - cxi * twi3).reshape(N2 * _BT, N1)
    dxi = (cxr * twi3 + cxi * twr3).reshape(N2 * _BT, N1)
    dyr = (cyr * twr3 - cyi * twi3).reshape(N2 * _BT, N1)
    dyi = (cyr * twi3 + cyi * twr3).reshape(N2 * _BT, N1)
    dpack = jnp.concatenate(
        [jnp.concatenate([dxr, dxi], axis=1),
         jnp.concatenate([dyr, dyi], axis=1)], axis=0)
    e = jnp.dot(dpack, f128p, preferred_element_type=jnp.float32)

    m = N2 * _BT
    eb = e.astype(bf16)
    xr, xi = eb[:m, :N1], eb[:m, N1:]
    yr, yi = eb[m:, :N1], eb[m:, N1:]
    zr = xr * yr - xi * yi
    zi = xr * yi + xi * yr

    zpack = jnp.concatenate([zr, zi], axis=1)
    p = jnp.dot(zpack, g128p_ref[...], preferred_element_type=jnp.float32)
    pb = p.astype(bf16).reshape(N2, _BT, 2 * N1)
    pr, pi = pb[:, :, :N1], pb[:, :, N1:]
    qr = (pr * twr3 + pi * twi3).reshape(N2, _BT * N1)
    qi = (pi * twr3 - pr * twi3).reshape(N2, _BT * N1)
    qpack = jnp.concatenate([qr, qi], axis=0)
    z3 = jnp.dot(g125p_ref[...], qpack,
                 preferred_element_type=jnp.float32) * jnp.float32(1.0 / N_OUT)
    out_ref[...] = z3.reshape(N2, _BT, N1).swapaxes(0, 1).reshape(_BT, N_OUT)


def _fftconv(xcs, ycs):
    nb = xcs.shape[0]
    consts = (_F125P, _F128P, _G128P, _G125P,
              _TWR.astype(jnp.bfloat16), _TWI.astype(jnp.bfloat16))
    const_specs = [
        pl.BlockSpec(c.shape, functools.partial(lambda n, i: (0,) * n, c.ndim))
        for c in consts
    ]
    return pl.pallas_call(
        _fftconv_body,
        grid=(nb // _BT,),
        in_specs=[
            pl.BlockSpec((_BT, N_OUT), lambda i: (i, 0)),
            pl.BlockSpec((_BT, N_OUT), lambda i: (i, 0)),
            *const_specs,
        ],
        out_specs=pl.BlockSpec((_BT, N_OUT), lambda i: (i, 0)),
        out_shape=jax.ShapeDtypeStruct((nb, N_OUT), jnp.float32),
    )(xcs, ycs, *(jnp.asarray(c) for c in consts))


_NW = 32
_NCHUNK = D_IN // 128


def _sc_sketch_pair(x, y, sign1, indx1, sign2, indx2):
    nb = x.shape[0]
    rows_per_w = nb // _NW
    mesh = plsc.VectorSubcoreMesh(core_axis_name="c", subcore_axis_name="s")
    out_sds = jax.ShapeDtypeStruct((nb, N_OUT), jnp.float32)

    @functools.partial(
        pl.kernel,
        out_type=(out_sds, out_sds),
        mesh=mesh,
        scratch_types=[
            pltpu.VMEM((D_IN,), jnp.float32),
            pltpu.VMEM((D_IN,), jnp.float32),
            pltpu.VMEM((D_IN,), jnp.float32),
            pltpu.VMEM((_NCHUNK, 128), jnp.int32),
            pltpu.VMEM((_NCHUNK, 128), jnp.int32),
            pltpu.VMEM((128,), jnp.float32),
            pltpu.VMEM_SHARED((16 * N_OUT,), jnp.float32),
            pltpu.SemaphoreType.DMA,
        ],
    )
    def sketch(x_hbm, y_hbm, s1_hbm, s2_hbm, i1_hbm, i2_hbm,
               xcs_hbm, ycs_hbm, xs_v, s1_v, s2_v, i1_v, i2_v, z_v, acc_sh,
               sem_z):
        sid = lax.axis_index("s")
        wid = sid * 2 + lax.axis_index("c")
        base = wid * rows_per_w
        abase = sid * N_OUT

        pltpu.sync_copy(s1_hbm, s1_v)
        pltpu.sync_copy(s2_hbm, s2_v)
        pltpu.sync_copy(i1_hbm, i1_v)
        pltpu.sync_copy(i2_hbm, i2_v)

        @pl.loop(0, _NCHUNK)
        def _(j):
            @pl.loop(0, 128, step=16)
            def _(c):
                i1_v[j, pl.ds(c, 16)] = i1_v[j, pl.ds(c, 16)] + abase
                i2_v[j, pl.ds(c, 16)] = i2_v[j, pl.ds(c, 16)] + abase

        @pl.loop(0, 128, step=16)
        def _(c):
            z_v[pl.ds(c, 16)] = jnp.zeros((16,), jnp.float32)

        @pl.loop(0, D_IN, step=16)
        def _(c):
            xs_v[pl.ds(c, 16)] = jnp.zeros((16,), jnp.float32)

        @pl.loop(0, N_OUT, step=D_IN)
        def _(c):
            pltpu.sync_copy(xs_v, acc_sh.at[pl.ds(abase + c, D_IN)])

        def one_sketch(row_hbm, sgn_v, idx_v, out_hbm, b):
            pltpu.sync_copy(row_hbm.at[b], xs_v)

            @pl.loop(0, D_IN, step=16)
            def _(c):
                xs_v[pl.ds(c, 16)] = xs_v[pl.ds(c, 16)] * sgn_v[pl.ds(c, 16)]

            @pl.loop(0, _NCHUNK)
            def _(j):
                pltpu.sync_copy(xs_v.at[pl.ds(j * 128, 128)],
                                acc_sh.at[idx_v.at[j]], add=True)

            pltpu.sync_copy(acc_sh.at[pl.ds(abase, N_OUT)], out_hbm.at[b])

            @pl.loop(0, _NCHUNK)
            def _(j):
                pltpu.async_copy(z_v, acc_sh.at[idx_v.at[j]], sem_z)

            @pl.loop(0, _NCHUNK)
            def _(j):
                pltpu.make_async_copy(z_v, acc_sh.at[idx_v.at[j]],
                                      sem_z).wait()

        @pl.loop(0, rows_per_w)
        def _(i):
            b = base + i
            one_sketch(x_hbm, s1_v, i1_v, xcs_hbm, b)
            one_sketch(y_hbm, s2_v, i2_v, ycs_hbm, b)

    return sketch(x, y, sign1, sign2,
                  indx1.reshape(_NCHUNK, 128), indx2.reshape(_NCHUNK, 128))


_NCHUNKS_B = 8


@jax.jit
def kernel(x, y, sign1, indx1, sign2, indx2):
    cb = BATCH // _NCHUNKS_B
    outs = []
    for c in range(_NCHUNKS_B):
        sl = slice(c * cb, (c + 1) * cb)
        xcs, ycs = _sc_sketch_pair(x[sl], y[sl], sign1, indx1, sign2, indx2)
        outs.append(_fftconv(xcs, ycs))
    return jnp.concatenate(outs, axis=0)
